# baseline (speedup 1.0000x reference)
import functools

import jax
import jax.numpy as jnp
from jax.experimental import pallas as pl
from jax.experimental.pallas import tpu as pltpu


def _proj_kernel(x_ref, y_ref, th_ref, xp_ref, yp_ref):
    th = th_ref[...]
    norm = jnp.sqrt(jnp.sum(th * th, axis=0, keepdims=True))
    thn = th / (norm + 1e-12)
    dims = (((0,), (1,)), ((), ()))
    xp_ref[...] = jax.lax.dot_general(
        thn, x_ref[...], dims, preferred_element_type=jnp.float32)
    yp_ref[...] = jax.lax.dot_general(
        thn, y_ref[...], dims, preferred_element_type=jnp.float32)


def _bitonic_sort_lanes(a, n):
    log_n = n.bit_length() - 1
    total = log_n * (log_n + 1) // 2
    i = jax.lax.broadcasted_iota(jnp.int32, (1, n), 1)

    def _pass(t, carry):
        a, k, j = carry
        asc = (i & k) == 0
        bit_lo = (i & j) == 0
        fwd = pltpu.roll(a, n - j, axis=1)
        bwd = pltpu.roll(a, j, axis=1)
        lo_val = jnp.where(asc, jnp.minimum(a, fwd), jnp.maximum(a, fwd))
        hi_val = jnp.where(asc, jnp.maximum(a, bwd), jnp.minimum(a, bwd))
        a = jnp.where(bit_lo, lo_val, hi_val)
        last = j == 1
        k_next = jnp.where(last, k * 2, k)
        j_next = jnp.where(last, k, j // 2)
        return (a, k_next, j_next)

    a, _, _ = jax.lax.fori_loop(
        0, total, _pass, (a, jnp.int32(2), jnp.int32(1)), unroll=False)
    return a


def _sort_kernel(xp_ref, out_ref, *, n):
    out_ref[...] = _bitonic_sort_lanes(xp_ref[...], n)


def _sort_diff_kernel(yp_ref, xs_ref, out_ref, *, n):
    ys = _bitonic_sort_lanes(yp_ref[...], n)
    s = jnp.sum(jnp.abs(xs_ref[...] - ys))
    out_ref[...] = jnp.broadcast_to(s, out_ref.shape)


def kernel(x, y, theta):
    n, d = x.shape
    l = theta.shape[1]
    group = 8
    num_groups = l // group

    row_block = 8192
    xp, yp = pl.pallas_call(
        _proj_kernel,
        grid=(n // row_block,),
        in_specs=[
            pl.BlockSpec((row_block, d), lambda i: (i, 0)),
            pl.BlockSpec((row_block, d), lambda i: (i, 0)),
            pl.BlockSpec((d, l), lambda i: (0, 0)),
        ],
        out_specs=[
            pl.BlockSpec((l, row_block), lambda i: (0, i)),
            pl.BlockSpec((l, row_block), lambda i: (0, i)),
        ],
        out_shape=[
            jax.ShapeDtypeStruct((l, n), jnp.float32),
            jax.ShapeDtypeStruct((l, n), jnp.float32),
        ],
    )(x, y, theta)

    xs = pl.pallas_call(
        functools.partial(_sort_kernel, n=n),
        grid=(num_groups,),
        in_specs=[pl.BlockSpec((group, n), lambda g: (g, 0))],
        out_specs=pl.BlockSpec((group, n), lambda g: (g, 0)),
        out_shape=jax.ShapeDtypeStruct((l, n), jnp.float32),
    )(xp)

    partial = pl.pallas_call(
        functools.partial(_sort_diff_kernel, n=n),
        grid=(num_groups,),
        in_specs=[
            pl.BlockSpec((group, n), lambda g: (g, 0)),
            pl.BlockSpec((group, n), lambda g: (g, 0)),
        ],
        out_specs=pl.BlockSpec((1, 8, 128), lambda g: (g, 0, 0)),
        out_shape=jax.ShapeDtypeStruct((num_groups, 8, 128), jnp.float32),
    )(yp, xs)

    return jnp.sum(partial[:, 0, 0]) / (n * l)
